# baseline (speedup 1.0000x reference)
import functools

import jax
import jax.numpy as jnp
from jax import lax
from jax.experimental import pallas as pl
from jax.experimental.pallas import tpu as pltpu
from jax.experimental.pallas import tpu_sc as plsc

_NCHUNKS = 16
_NBUF = 4


def kernel(x, emb):
    seq_len = x.shape[1]
    d = emb.shape[1]
    info = plsc.get_sparse_core_info()
    nc, ns = info.num_cores, info.num_subcores
    rows_w = seq_len // (nc * ns)
    c = rows_w // _NCHUNKS
    mesh = plsc.VectorSubcoreMesh(core_axis_name="c", subcore_axis_name="s")

    @functools.partial(
        pl.kernel,
        out_type=jax.ShapeDtypeStruct((seq_len, d), emb.dtype),
        mesh=mesh,
        scratch_types=[
            pltpu.VMEM((_NBUF, c, d), jnp.float32),
            pltpu.SemaphoreType.DMA((_NBUF,)),
            pltpu.SemaphoreType.DMA((_NBUF,)),
        ],
    )
    def run(emb_hbm, out_hbm, buf, rsems, wsems):
        wid = lax.axis_index("s") * nc + lax.axis_index("c")
        base = wid * rows_w

        def rd(i):
            return pltpu.make_async_copy(
                emb_hbm.at[pl.ds(base + i * c, c)],
                buf.at[i % _NBUF],
                rsems.at[i % _NBUF],
            )

        def wr(i):
            return pltpu.make_async_copy(
                buf.at[i % _NBUF],
                out_hbm.at[pl.ds(base + i * c, c)],
                wsems.at[i % _NBUF],
            )

        for i in range(_NBUF):
            rd(i).start()
        for i in range(_NCHUNKS):
            rd(i).wait()
            wr(i).start()
            if i + _NBUF < _NCHUNKS:
                wr(i).wait()
                rd(i + _NBUF).start()
        for i in range(_NCHUNKS - _NBUF, _NCHUNKS):
            wr(i).wait()

    return run(emb)


# speedup vs baseline: 2.0075x; 2.0075x over previous
import jax
import jax.numpy as jnp
from jax.experimental import pallas as pl
from jax.experimental.pallas import tpu as pltpu

_NCHUNKS = 8


def _copy_body(emb_ref, out_ref, buf, rsems, wsems):
    rows = emb_ref.shape[0]
    c = rows // _NCHUNKS

    def rd(i):
        return pltpu.make_async_copy(
            emb_ref.at[pl.ds(i * c, c)], buf.at[i], rsems.at[i])

    def wr(i):
        return pltpu.make_async_copy(
            buf.at[i], out_ref.at[pl.ds(i * c, c)], wsems.at[i])

    for i in range(_NCHUNKS):
        rd(i).start()
    for i in range(_NCHUNKS):
        rd(i).wait()
        wr(i).start()
    for i in range(_NCHUNKS):
        wr(i).wait()


def kernel(x, emb):
    seq_len = x.shape[1]
    d = emb.shape[1]
    c = seq_len // _NCHUNKS
    return pl.pallas_call(
        _copy_body,
        in_specs=[pl.BlockSpec(memory_space=pl.ANY)],
        out_specs=pl.BlockSpec(memory_space=pl.ANY),
        out_shape=jax.ShapeDtypeStruct((seq_len, d), emb.dtype),
        scratch_shapes=[
            pltpu.VMEM((_NCHUNKS, c, d), emb.dtype),
            pltpu.SemaphoreType.DMA((_NCHUNKS,)),
            pltpu.SemaphoreType.DMA((_NCHUNKS,)),
        ],
    )(emb)
